# baseline (speedup 1.0000x reference)
import jax
import jax.numpy as jnp
from jax import lax
from jax.experimental import pallas as pl
from jax.experimental.pallas import tpu as pltpu
from jax.experimental.pallas import tpu_sc as plsc

D_MODEL = 1024
ROWS = 4 * 4096
NUM_SEG = 2
LANES = 16

NUM_CORES = 2
NUM_SUBCORES = 16
NUM_WORKERS = 32
ROWS_PER_W = ROWS // NUM_WORKERS
CHUNK_R = 8
NCHUNK = ROWS_PER_W // CHUNK_R
NBUF = 8
PREF = 3


def _body(x_hbm, sid_hbm, emb_hbm, out_hbm, sid_v, emb_v, buf, sem_in,
          sem_out, sem_stage):
  wid = lax.axis_index("s") * NUM_CORES + lax.axis_index("c")
  base = wid * ROWS_PER_W

  def start_in(c, b):
    pltpu.async_copy(
        x_hbm.at[pl.ds(base + c * CHUNK_R, CHUNK_R), :], buf[b], sem_in[b])

  def wait_in(b):
    pltpu.make_async_copy(
        x_hbm.at[pl.ds(base, CHUNK_R), :], buf[b], sem_in[b]).wait()

  def start_out(c, b):
    pltpu.async_copy(
        buf[b], out_hbm.at[pl.ds(base + c * CHUNK_R, CHUNK_R), :],
        sem_out[b])

  def wait_out(b):
    pltpu.make_async_copy(
        buf[b], out_hbm.at[pl.ds(base, CHUNK_R), :], sem_out[b]).wait()

  def compute(c, b):
    xr = buf[b]
    crow = c * CHUNK_R

    def row_body(r, carry):
      sid_s = sid_v[pl.ds(crow + r, LANES)][0]
      off = sid_s * D_MODEL

      @plsc.parallel_loop(0, D_MODEL // LANES, step=1, unroll=16)
      def jbody(j):
        plsc.addupdate(xr.at[r, pl.ds(j * LANES, LANES)],
                       emb_v[pl.ds(off + j * LANES, LANES)])

      return carry

    lax.fori_loop(0, CHUNK_R, row_body, 0, unroll=False)

  for b in range(NBUF):
    start_in(b, b)

  h_emb = pltpu.async_copy(emb_hbm, emb_v, sem_stage)
  h_sid = pltpu.async_copy(sid_hbm.at[pl.ds(base, ROWS_PER_W)],
                           sid_v.at[pl.ds(0, ROWS_PER_W)], sem_stage)
  h_emb.wait()
  h_sid.wait()

  @pl.loop(0, NCHUNK, step=NBUF)
  def chunk_loop(g):
    for b in range(NBUF):
      c = g + b
      b2 = (b + PREF) % NBUF

      @pl.when(jnp.logical_and(c >= NBUF - PREF, c + PREF < NCHUNK))
      def _():
        wait_out(b2)
        start_in(c + PREF, b2)

      wait_in(b)
      compute(c, b)
      start_out(c, b)

  for b in range(NBUF):
    wait_out(b)


@jax.jit
def _run(x2, sid, emb):
  mesh = plsc.VectorSubcoreMesh(
      core_axis_name="c", subcore_axis_name="s",
      num_cores=NUM_CORES, num_subcores=NUM_SUBCORES)
  f = pl.kernel(
      _body,
      out_type=jax.ShapeDtypeStruct((ROWS, D_MODEL), jnp.float32),
      mesh=mesh,
      compiler_params=pltpu.CompilerParams(needs_layout_passes=False),
      scratch_types=[
          pltpu.VMEM((ROWS_PER_W + LANES,), jnp.int32),
          pltpu.VMEM((NUM_SEG * D_MODEL,), jnp.float32),
          [pltpu.VMEM((CHUNK_R, D_MODEL), jnp.float32) for _ in range(NBUF)],
          [pltpu.SemaphoreType.DMA for _ in range(NBUF)],
          [pltpu.SemaphoreType.DMA for _ in range(NBUF)],
          pltpu.SemaphoreType.DMA,
      ],
  )
  return f(x2, sid, emb)


def kernel(x, segment_ids, seg_emb):
  b, s, d = x.shape
  x2 = x.reshape(b * s, d)
  sid = segment_ids.reshape(b * s).astype(jnp.int32)
  out = _run(x2, sid, seg_emb.reshape(NUM_SEG * D_MODEL))
  return out.reshape(b, s, d)


# speedup vs baseline: 1.0015x; 1.0015x over previous
import jax
import jax.numpy as jnp
from jax import lax
from jax.experimental import pallas as pl
from jax.experimental.pallas import tpu as pltpu
from jax.experimental.pallas import tpu_sc as plsc

D_MODEL = 1024
ROWS = 4 * 4096
NUM_SEG = 2
LANES = 16

NUM_CORES = 2
NUM_SUBCORES = 16
NUM_WORKERS = 32
ROWS_PER_W = ROWS // NUM_WORKERS
CHUNK_R = 16
NCHUNK = ROWS_PER_W // CHUNK_R
NBUF = 4
PREF = 2
HALF_R = CHUNK_R // 2


def _body(x_hbm, sid_hbm, emb_hbm, out_hbm, sid_v, emb_v, buf, sem_in,
          sem_out, sem_stage):
  wid = lax.axis_index("s") * NUM_CORES + lax.axis_index("c")
  base = wid * ROWS_PER_W

  def start_in(c, b):
    pltpu.async_copy(
        x_hbm.at[pl.ds(base + c * CHUNK_R, CHUNK_R), :], buf[b], sem_in[b])

  def wait_in(b):
    pltpu.make_async_copy(
        x_hbm.at[pl.ds(base, CHUNK_R), :], buf[b], sem_in[b]).wait()

  def start_out_half(c, b, half):
    pltpu.async_copy(
        buf[b].at[pl.ds(half * HALF_R, HALF_R), :],
        out_hbm.at[pl.ds(base + c * CHUNK_R + half * HALF_R, HALF_R), :],
        sem_out[b])

  def wait_out(b):
    pltpu.make_async_copy(
        buf[b], out_hbm.at[pl.ds(base, CHUNK_R), :], sem_out[b]).wait()

  def compute_rows(c, b, r0, r1):
    xr = buf[b]
    crow = c * CHUNK_R

    def row_body(r, carry):
      sid_s = sid_v[pl.ds(crow + r, LANES)][0]
      off = sid_s * D_MODEL

      @plsc.parallel_loop(0, D_MODEL // LANES, step=1, unroll=16)
      def jbody(j):
        plsc.addupdate(xr.at[r, pl.ds(j * LANES, LANES)],
                       emb_v[pl.ds(off + j * LANES, LANES)])

      return carry

    lax.fori_loop(r0, r1, row_body, 0, unroll=False)

  for b in range(NBUF):
    start_in(b, b)

  h_emb = pltpu.async_copy(emb_hbm, emb_v, sem_stage)
  h_sid = pltpu.async_copy(sid_hbm.at[pl.ds(base, ROWS_PER_W)],
                           sid_v.at[pl.ds(0, ROWS_PER_W)], sem_stage)
  h_emb.wait()
  h_sid.wait()

  @pl.loop(0, NCHUNK, step=NBUF)
  def chunk_loop(g):
    for b in range(NBUF):
      c = g + b
      b2 = (b + PREF) % NBUF

      @pl.when(jnp.logical_and(c >= NBUF - PREF, c + PREF < NCHUNK))
      def _():
        wait_out(b2)
        start_in(c + PREF, b2)

      wait_in(b)
      compute_rows(c, b, 0, HALF_R)
      start_out_half(c, b, 0)
      compute_rows(c, b, HALF_R, CHUNK_R)
      start_out_half(c, b, 1)

  for b in range(NBUF):
    wait_out(b)


@jax.jit
def _run(x2, sid, emb):
  mesh = plsc.VectorSubcoreMesh(
      core_axis_name="c", subcore_axis_name="s",
      num_cores=NUM_CORES, num_subcores=NUM_SUBCORES)
  f = pl.kernel(
      _body,
      out_type=jax.ShapeDtypeStruct((ROWS, D_MODEL), jnp.float32),
      mesh=mesh,
      compiler_params=pltpu.CompilerParams(needs_layout_passes=False),
      scratch_types=[
          pltpu.VMEM((ROWS_PER_W + LANES,), jnp.int32),
          pltpu.VMEM((NUM_SEG * D_MODEL,), jnp.float32),
          [pltpu.VMEM((CHUNK_R, D_MODEL), jnp.float32) for _ in range(NBUF)],
          [pltpu.SemaphoreType.DMA for _ in range(NBUF)],
          [pltpu.SemaphoreType.DMA for _ in range(NBUF)],
          pltpu.SemaphoreType.DMA,
      ],
  )
  return f(x2, sid, emb)


def kernel(x, segment_ids, seg_emb):
  b, s, d = x.shape
  x2 = x.reshape(b * s, d)
  sid = segment_ids.reshape(b * s).astype(jnp.int32)
  out = _run(x2, sid, seg_emb.reshape(NUM_SEG * D_MODEL))
  return out.reshape(b, s, d)


# speedup vs baseline: 1.0114x; 1.0099x over previous
import jax
import jax.numpy as jnp
from jax import lax
from jax.experimental import pallas as pl
from jax.experimental.pallas import tpu as pltpu
from jax.experimental.pallas import tpu_sc as plsc

D_MODEL = 1024
ROWS = 4 * 4096
NUM_SEG = 2
LANES = 16

NUM_CORES = 2
NUM_SUBCORES = 16
NUM_WORKERS = 32
ROWS_PER_W = ROWS // NUM_WORKERS
CHUNK_R = 16
NCHUNK = ROWS_PER_W // CHUNK_R
NBUF = 4


def _body(x_hbm, sid_hbm, emb_hbm, out_hbm, sid_v, emb_v, buf, sem_in,
          sem_out, sem_stage):
  wid = lax.axis_index("s") * NUM_CORES + lax.axis_index("c")
  base = wid * ROWS_PER_W

  def start_in(c, b):
    pltpu.async_copy(
        x_hbm.at[pl.ds(base + c * CHUNK_R, CHUNK_R), :], buf[b], sem_in[b])

  def wait_in(b):
    pltpu.make_async_copy(
        x_hbm.at[pl.ds(base, CHUNK_R), :], buf[b], sem_in[b]).wait()

  def start_out(c, b):
    pltpu.async_copy(
        buf[b], out_hbm.at[pl.ds(base + c * CHUNK_R, CHUNK_R), :],
        sem_out[b])

  def wait_out(b):
    pltpu.make_async_copy(
        buf[b], out_hbm.at[pl.ds(base, CHUNK_R), :], sem_out[b]).wait()

  def compute(c, b):
    xr = buf[b]
    crow = c * CHUNK_R

    def row_body(r, carry):
      sid_s = sid_v[pl.ds(crow + r, LANES)][0]
      off = sid_s * D_MODEL

      @plsc.parallel_loop(0, D_MODEL // LANES, step=1, unroll=16)
      def jbody(j):
        plsc.addupdate(xr.at[r, pl.ds(j * LANES, LANES)],
                       emb_v[pl.ds(off + j * LANES, LANES)])

      return carry

    lax.fori_loop(0, CHUNK_R, row_body, 0, unroll=False)

  for b in range(NBUF):
    start_in(b, b)

  h_emb = pltpu.async_copy(emb_hbm, emb_v, sem_stage)
  h_sid = pltpu.async_copy(sid_hbm.at[pl.ds(base, ROWS_PER_W)],
                           sid_v.at[pl.ds(0, ROWS_PER_W)], sem_stage)
  h_emb.wait()
  h_sid.wait()

  @pl.loop(0, NCHUNK, step=NBUF)
  def chunk_loop(g):
    for b in range(NBUF):
      c = g + b
      b2 = (b + 2) % NBUF

      @pl.when(jnp.logical_and(c >= 2, c + 2 < NCHUNK))
      def _():
        wait_out(b2)
        start_in(c + 2, b2)

      wait_in(b)
      compute(c, b)
      start_out(c, b)

  for b in range(NBUF):
    wait_out(b)


@jax.jit
def _run(x2, sid, emb):
  mesh = plsc.VectorSubcoreMesh(
      core_axis_name="c", subcore_axis_name="s",
      num_cores=NUM_CORES, num_subcores=NUM_SUBCORES)
  f = pl.kernel(
      _body,
      out_type=jax.ShapeDtypeStruct((ROWS, D_MODEL), jnp.float32),
      mesh=mesh,
      compiler_params=pltpu.CompilerParams(needs_layout_passes=False),
      scratch_types=[
          pltpu.VMEM((ROWS_PER_W + LANES,), jnp.int32),
          pltpu.VMEM((NUM_SEG * D_MODEL,), jnp.float32),
          [pltpu.VMEM((CHUNK_R, D_MODEL), jnp.float32) for _ in range(NBUF)],
          [pltpu.SemaphoreType.DMA for _ in range(NBUF)],
          [pltpu.SemaphoreType.DMA for _ in range(NBUF)],
          pltpu.SemaphoreType.DMA,
      ],
  )
  return f(x2, sid, emb)


def kernel(x, segment_ids, seg_emb):
  b, s, d = x.shape
  x2 = x.reshape(b * s, d)
  sid = segment_ids.reshape(b * s).astype(jnp.int32)
  out = _run(x2, sid, seg_emb.reshape(NUM_SEG * D_MODEL))
  return out.reshape(b, s, d)
